# baseline (speedup 1.0000x reference)
import functools

import jax
import jax.numpy as jnp
from jax import lax
from jax.experimental import pallas as pl
from jax.experimental.pallas import tpu as pltpu
from jax.experimental.pallas import tpu_sc as plsc

E = 160000
N = 10000
P = 50
PP = 56
NPAD = 10240
BLK = 128
FLW = BLK * P
NTILES = 32
EBLKS = 39
ETAIL = EBLKS * BLK * NTILES
OPAD = NTILES * 3 * BLK


def _sc_scatter_body(eflat, eidx, oflat, oidx, zeros, out,
                     fb, pb, idxb, acc, sem0, sem1):
    c = lax.axis_index("c")
    s = lax.axis_index("s")
    wid = c * 16 + s

    pltpu.sync_copy(zeros, acc.at[pl.ds(s * 640, 640)])
    plsc.subcore_barrier()

    sems = (sem0, sem1)

    def start(b, midx_hbm, mflat_hbm, base):
        pltpu.async_copy(midx_hbm.at[pl.ds(base, BLK)], idxb.at[b], sems[b])
        pltpu.async_copy(mflat_hbm.at[pl.ds(base * P, FLW)],
                         fb.at[b, pl.ds(0, FLW)], sems[b])

    def wait_scat(b):
        pltpu.make_async_copy(eidx.at[pl.ds(0, BLK)], idxb.at[b],
                              sems[b]).wait()
        pltpu.make_async_copy(eflat.at[pl.ds(0, FLW)],
                              fb.at[b, pl.ds(0, FLW)], sems[b]).wait()

        def row(r, carry):
            src = r * P
            pb[b, r, pl.ds(0, 16)] = fb[b, pl.ds(src, 16)]
            pb[b, r, pl.ds(16, 16)] = fb[b, pl.ds(src + 16, 16)]
            pb[b, r, pl.ds(32, 16)] = fb[b, pl.ds(src + 32, 16)]
            pb[b, r, pl.ds(40, 16)] = fb[b, pl.ds(src + 40, 16)]
            return carry

        lax.fori_loop(0, BLK, row, 0)
        pltpu.sync_copy(pb.at[b], acc.at[idxb.at[b]], add=True)

    ebase = wid * (EBLKS * BLK)

    start(0, eidx, eflat, ebase)

    def eloop(k, carry):
        start(1, eidx, eflat, ebase + (2 * k + 1) * BLK)
        wait_scat(0)
        start(0, eidx, eflat, ebase + (2 * k + 2) * BLK)
        wait_scat(1)
        return carry

    lax.fori_loop(0, (EBLKS - 1) // 2, eloop, 0)
    wait_scat(0)

    @pl.when(wid == 0)
    def _():
        start(0, eidx, eflat, ETAIL)
        wait_scat(0)

    @pl.when(wid == 1)
    def _():
        start(0, eidx, eflat, ETAIL + BLK)
        wait_scat(0)

    obase = wid * (3 * BLK)
    start(0, oidx, oflat, obase)
    start(1, oidx, oflat, obase + BLK)
    wait_scat(0)
    start(0, oidx, oflat, obase + 2 * BLK)
    wait_scat(1)
    wait_scat(0)

    plsc.subcore_barrier()

    pltpu.sync_copy(acc.at[pl.ds(s * 640, 640)],
                    out.at[c, pl.ds(s * 640, 640)])


_sc_scatter = functools.partial(
    pl.kernel,
    out_type=jax.ShapeDtypeStruct((2, NPAD, PP), jnp.float32),
    mesh=plsc.VectorSubcoreMesh(core_axis_name="c", subcore_axis_name="s"),
    compiler_params=pltpu.CompilerParams(use_tc_tiling_on_sc=False),
    scratch_types=[
        pltpu.VMEM((2, FLW + 16), jnp.float32),
        pltpu.VMEM((2, BLK, PP), jnp.float32),
        pltpu.VMEM((2, BLK), jnp.int32),
        pltpu.VMEM_SHARED((NPAD, PP), jnp.float32),
        pltpu.SemaphoreType.DMA,
        pltpu.SemaphoreType.DMA,
    ],
)(_sc_scatter_body)


def _norm_body(p_ref, o_ref):
    x = p_ref[0, :, :P] + p_ref[1, :, :P]
    m = jnp.max(x, axis=-1, keepdims=True)
    e = jnp.exp(x - m)
    lse = jnp.log(jnp.sum(e, axis=-1, keepdims=True)) + m
    o_ref[...] = x - lse


def _normalize(part):
    return pl.pallas_call(
        _norm_body,
        out_shape=jax.ShapeDtypeStruct((NPAD, P), jnp.float32),
    )(part)


def kernel(m_w_j2i, m_w_o2i, e_idx, o_idx):
    eflat = m_w_j2i.reshape(E * P)
    npad = OPAD - (N - 1)
    oidx_p = jnp.concatenate(
        [o_idx, jnp.arange(npad, dtype=jnp.int32) % N])
    oflat = jnp.concatenate(
        [m_w_o2i.reshape((N - 1) * P),
         jnp.zeros(npad * P, jnp.float32)])
    zeros = jnp.zeros((640, PP), jnp.float32)
    part = _sc_scatter(eflat, e_idx, oflat, oidx_p, zeros)
    out = _normalize(part)
    return out[:N].reshape(1, N, P, 1)


# speedup vs baseline: 1.4824x; 1.4824x over previous
import functools

import jax
import jax.numpy as jnp
from jax import lax
from jax.experimental import pallas as pl
from jax.experimental.pallas import tpu as pltpu
from jax.experimental.pallas import tpu_sc as plsc

E = 160000
N = 10000
P = 50
PP = 128
NPAD = 10240
BLK = 128
NTILES = 32
EBLKS = 39
ETAIL = EBLKS * BLK * NTILES
OPAD = NTILES * 3 * BLK


def _sc_scatter_body(emsg, eidx, omsg, oidx, zeros, out,
                     msgb, idxb, acc, sem0, sem1):
    c = lax.axis_index("c")
    s = lax.axis_index("s")
    wid = c * 16 + s

    pltpu.sync_copy(zeros, acc.at[pl.ds(s * 640, 640)])
    plsc.subcore_barrier()

    sems = (sem0, sem1)

    def start(b, midx_hbm, mmsg_hbm, base):
        pltpu.async_copy(midx_hbm.at[pl.ds(base, BLK)], idxb.at[b], sems[b])
        pltpu.async_copy(mmsg_hbm.at[pl.ds(base, BLK)], msgb.at[b], sems[b])

    def wait_scat(b):
        pltpu.make_async_copy(eidx.at[pl.ds(0, BLK)], idxb.at[b],
                              sems[b]).wait()
        pltpu.make_async_copy(emsg.at[pl.ds(0, BLK)], msgb.at[b],
                              sems[b]).wait()
        pltpu.sync_copy(msgb.at[b], acc.at[idxb.at[b]], add=True)

    ebase = wid * (EBLKS * BLK)

    start(0, eidx, emsg, ebase)

    def eloop(k, carry):
        start(1, eidx, emsg, ebase + (2 * k + 1) * BLK)
        wait_scat(0)
        start(0, eidx, emsg, ebase + (2 * k + 2) * BLK)
        wait_scat(1)
        return carry

    lax.fori_loop(0, (EBLKS - 1) // 2, eloop, 0)
    wait_scat(0)

    @pl.when(wid == 0)
    def _():
        start(0, eidx, emsg, ETAIL)
        wait_scat(0)

    @pl.when(wid == 1)
    def _():
        start(0, eidx, emsg, ETAIL + BLK)
        wait_scat(0)

    obase = wid * (3 * BLK)
    start(0, oidx, omsg, obase)
    start(1, oidx, omsg, obase + BLK)
    wait_scat(0)
    start(0, oidx, omsg, obase + 2 * BLK)
    wait_scat(1)
    wait_scat(0)

    plsc.subcore_barrier()

    pltpu.sync_copy(acc.at[pl.ds(s * 640, 640)],
                    out.at[c, pl.ds(s * 640, 640)])


_sc_scatter = functools.partial(
    pl.kernel,
    out_type=jax.ShapeDtypeStruct((2, NPAD, PP), jnp.float32),
    mesh=plsc.VectorSubcoreMesh(core_axis_name="c", subcore_axis_name="s"),
    compiler_params=pltpu.CompilerParams(use_tc_tiling_on_sc=False),
    scratch_types=[
        pltpu.VMEM((2, BLK, PP), jnp.float32),
        pltpu.VMEM((2, BLK), jnp.int32),
        pltpu.VMEM_SHARED((NPAD, PP), jnp.float32),
        pltpu.SemaphoreType.DMA,
        pltpu.SemaphoreType.DMA,
    ],
)(_sc_scatter_body)


def _norm_body(p_ref, o_ref):
    x = p_ref[0, :, :P] + p_ref[1, :, :P]
    m = jnp.max(x, axis=-1, keepdims=True)
    e = jnp.exp(x - m)
    lse = jnp.log(jnp.sum(e, axis=-1, keepdims=True)) + m
    o_ref[...] = x - lse


def _normalize(part):
    return pl.pallas_call(
        _norm_body,
        out_shape=jax.ShapeDtypeStruct((NPAD, P), jnp.float32),
    )(part)


def kernel(m_w_j2i, m_w_o2i, e_idx, o_idx):
    emsg = jnp.pad(m_w_j2i.reshape(E, P), ((0, 0), (0, PP - P)))
    npad = OPAD - (N - 1)
    oidx_p = jnp.concatenate(
        [o_idx, jnp.arange(npad, dtype=jnp.int32) % N])
    omsg_p = jnp.pad(m_w_o2i.reshape(N - 1, P),
                     ((0, npad), (0, PP - P)))
    zeros = jnp.zeros((640, PP), jnp.float32)
    part = _sc_scatter(emsg, e_idx, omsg_p, oidx_p, zeros)
    out = _normalize(part)
    return out[:N].reshape(1, N, P, 1)
